# initial kernel scaffold (unmeasured)
import jax
import jax.numpy as jnp
from jax import lax
from jax.experimental import pallas as pl
from jax.experimental.pallas import tpu as pltpu


def kernel(
    x,
):
    def body(*refs):
        pass

    out_shape = jax.ShapeDtypeStruct(..., jnp.float32)
    return pl.pallas_call(body, out_shape=out_shape)(...)


# baseline (device time: 865969 ns/iter reference)
import jax
import jax.numpy as jnp
from jax import lax
from jax.experimental import pallas as pl
from jax.experimental.pallas import tpu as pltpu

CHUNK = 1024


def kernel(x):
    m, n = x.shape

    def body(x_ref, out_ref, recv_ref, xa_ref, acc_ref, copy_sems, send_sem, recv_sem):
        my_x = lax.axis_index("x")
        my_y = lax.axis_index("y")
        my_z = lax.axis_index("z")

        rdma = pltpu.make_async_remote_copy(
            src_ref=x_ref,
            dst_ref=recv_ref,
            send_sem=send_sem,
            recv_sem=recv_sem,
            device_id=(1 - my_x, my_y, my_z),
            device_id_type=pl.DeviceIdType.MESH,
        )
        rdma.start()
        rdma.wait()

        for c in range(m // CHUNK):
            sl = pl.ds(c * CHUNK, CHUNK)
            cp_x = pltpu.make_async_copy(x_ref.at[sl], xa_ref, copy_sems.at[0])
            cp_r = pltpu.make_async_copy(recv_ref.at[sl], acc_ref, copy_sems.at[1])
            cp_x.start()
            cp_r.start()
            cp_x.wait()
            cp_r.wait()
            acc_ref[...] = acc_ref[...] + xa_ref[...]
            cp_o = pltpu.make_async_copy(acc_ref, out_ref.at[sl], copy_sems.at[2])
            cp_o.start()
            cp_o.wait()

    out, _ = pl.pallas_call(
        body,
        out_shape=[
            jax.ShapeDtypeStruct((m, n), jnp.float32),
            jax.ShapeDtypeStruct((m, n), jnp.float32),
        ],
        in_specs=[pl.BlockSpec(memory_space=pltpu.MemorySpace.HBM)],
        out_specs=[
            pl.BlockSpec(memory_space=pltpu.MemorySpace.HBM),
            pl.BlockSpec(memory_space=pltpu.MemorySpace.HBM),
        ],
        scratch_shapes=[
            pltpu.VMEM((CHUNK, n), jnp.float32),
            pltpu.VMEM((CHUNK, n), jnp.float32),
            pltpu.SemaphoreType.DMA((3,)),
            pltpu.SemaphoreType.DMA,
            pltpu.SemaphoreType.DMA,
        ],
    )(x)
    return out


# device time: 434530 ns/iter; 1.9929x vs baseline; 1.9929x over previous
import jax
import jax.numpy as jnp
from jax import lax
from jax.experimental import pallas as pl
from jax.experimental.pallas import tpu as pltpu

CHUNK = 1024


def kernel(x):
    m, n = x.shape
    qsize = m // 4

    def body(x_ref, out_ref, recv_ref, xa_ref, acc_ref, copy_sems,
             p1_send, p1_recv, p2_send, p2_recv):
        my_x = lax.axis_index("x")
        my_y = lax.axis_index("y")
        my_z = lax.axis_index("z")
        q = 2 * (my_x ^ my_y) + (my_x ^ my_z)
        x_nbr = (1 - my_x, my_y, my_z)
        y_nbr = (my_x, 1 - my_y, my_z)
        z_nbr = (my_x, my_y, 1 - my_z)

        qn = q ^ 3
        p1 = pltpu.make_async_remote_copy(
            src_ref=x_ref.at[pl.ds(qn * qsize, qsize)],
            dst_ref=recv_ref,
            send_sem=p1_send,
            recv_sem=p1_recv,
            device_id=x_nbr,
            device_id_type=pl.DeviceIdType.MESH,
        )
        p1.start()
        p1.wait()

        for c in range(qsize // CHUNK):
            cp_x = pltpu.make_async_copy(
                x_ref.at[pl.ds(q * qsize + c * CHUNK, CHUNK)],
                xa_ref, copy_sems.at[0])
            cp_x.start()
            cp_x.wait()
            acc_ref[...] = xa_ref[...] + recv_ref[pl.ds(c * CHUNK, CHUNK), :]
            cp_o = pltpu.make_async_copy(
                acc_ref,
                out_ref.at[pl.ds(q * qsize + c * CHUNK, CHUNK)],
                copy_sems.at[1])
            cp_o.start()
            cp_o.wait()

        my_slice = pl.ds(q * qsize, qsize)
        sends = []
        for k, nbr in enumerate((x_nbr, y_nbr, z_nbr)):
            s = pltpu.make_async_remote_copy(
                src_ref=out_ref.at[my_slice],
                dst_ref=out_ref.at[my_slice],
                send_sem=p2_send.at[k],
                recv_sem=p2_recv.at[k],
                device_id=nbr,
                device_id_type=pl.DeviceIdType.MESH,
            )
            s.start()
            sends.append(s)

        for k, (nbr, nq) in enumerate(((x_nbr, q ^ 3), (y_nbr, q ^ 2),
                                       (z_nbr, q ^ 1))):
            nbr_slice = pl.ds(nq * qsize, qsize)
            r = pltpu.make_async_remote_copy(
                src_ref=out_ref.at[nbr_slice],
                dst_ref=out_ref.at[nbr_slice],
                send_sem=p2_send.at[k],
                recv_sem=p2_recv.at[k],
                device_id=nbr,
                device_id_type=pl.DeviceIdType.MESH,
            )
            sends[k].wait_send()
            r.wait_recv()

    return pl.pallas_call(
        body,
        out_shape=jax.ShapeDtypeStruct((m, n), jnp.float32),
        in_specs=[pl.BlockSpec(memory_space=pltpu.MemorySpace.HBM)],
        out_specs=pl.BlockSpec(memory_space=pltpu.MemorySpace.HBM),
        scratch_shapes=[
            pltpu.VMEM((qsize, n), jnp.float32),
            pltpu.VMEM((CHUNK, n), jnp.float32),
            pltpu.VMEM((CHUNK, n), jnp.float32),
            pltpu.SemaphoreType.DMA((2,)),
            pltpu.SemaphoreType.DMA,
            pltpu.SemaphoreType.DMA,
            pltpu.SemaphoreType.DMA((3,)),
            pltpu.SemaphoreType.DMA((3,)),
        ],
    )(x)


# device time: 371958 ns/iter; 2.3281x vs baseline; 1.1682x over previous
import jax
import jax.numpy as jnp
from jax import lax
from jax.experimental import pallas as pl
from jax.experimental.pallas import tpu as pltpu

NC = 4


def kernel(x):
    m, n = x.shape
    qsize = m // 4
    ch = qsize // NC
    nc2 = NC // 2

    def body(x_ref, out_ref, recv_ref, xq_ref,
             p1s, p1r, dys, dyr, dzs, dzr, fys, fyr, fzs, fzr, sts, xls):
        my_x = lax.axis_index("x")
        my_y = lax.axis_index("y")
        my_z = lax.axis_index("z")
        r = 2 * my_y + my_z
        ry = r ^ 2
        rz = r ^ 1
        diag = r ^ 3
        base = r * qsize
        x_nbr = (1 - my_x, my_y, my_z)
        y_nbr = (my_x, 1 - my_y, my_z)
        z_nbr = (my_x, my_y, 1 - my_z)

        xq_load = pltpu.make_async_copy(
            x_ref.at[pl.ds(base, qsize)], xq_ref, xls)
        xq_load.start()

        p1 = []
        for c in range(NC):
            d = pltpu.make_async_remote_copy(
                src_ref=x_ref.at[pl.ds(base + c * ch, ch)],
                dst_ref=recv_ref.at[pl.ds(c * ch, ch)],
                send_sem=p1s.at[c],
                recv_sem=p1r.at[c],
                device_id=x_nbr,
                device_id_type=pl.DeviceIdType.MESH,
            )
            d.start()
            p1.append(d)

        xq_load.wait()

        st, dy, dz = [], [], []
        for c in range(NC):
            sl = pl.ds(c * ch, ch)
            p1[c].wait_recv()
            recv_ref[sl, :] = recv_ref[sl, :] + xq_ref[sl, :]
            s = pltpu.make_async_copy(
                recv_ref.at[sl], out_ref.at[pl.ds(base + c * ch, ch)],
                sts.at[c])
            s.start()
            st.append(s)
            for lst, sems_s, sems_r, nbr in (
                    (dy, dys, dyr, y_nbr), (dz, dzs, dzr, z_nbr)):
                d = pltpu.make_async_remote_copy(
                    src_ref=recv_ref.at[sl],
                    dst_ref=out_ref.at[pl.ds(base + c * ch, ch)],
                    send_sem=sems_s.at[c],
                    recv_sem=sems_r.at[c],
                    device_id=nbr,
                    device_id_type=pl.DeviceIdType.MESH,
                )
                d.start()
                lst.append(d)

        def recv_desc(qidx, c, sems_r):
            sl = pl.ds(qidx * qsize + c * ch, ch)
            return pltpu.make_async_remote_copy(
                src_ref=out_ref.at[sl], dst_ref=out_ref.at[sl],
                send_sem=p1s.at[0],
                recv_sem=sems_r,
                device_id=x_nbr,
                device_id_type=pl.DeviceIdType.MESH,
            )

        fwd = []
        for c in range(nc2):
            recv_desc(rz, c, dzr.at[c]).wait_recv()
            sl = pl.ds(rz * qsize + c * ch, ch)
            d = pltpu.make_async_remote_copy(
                src_ref=out_ref.at[sl], dst_ref=out_ref.at[sl],
                send_sem=fys.at[c], recv_sem=fyr.at[c],
                device_id=y_nbr, device_id_type=pl.DeviceIdType.MESH,
            )
            d.start()
            fwd.append(d)
        for c in range(nc2, NC):
            recv_desc(ry, c, dyr.at[c]).wait_recv()
            sl = pl.ds(ry * qsize + c * ch, ch)
            d = pltpu.make_async_remote_copy(
                src_ref=out_ref.at[sl], dst_ref=out_ref.at[sl],
                send_sem=fzs.at[c - nc2], recv_sem=fzr.at[c - nc2],
                device_id=z_nbr, device_id_type=pl.DeviceIdType.MESH,
            )
            d.start()
            fwd.append(d)

        for c in range(nc2, NC):
            recv_desc(rz, c, dzr.at[c]).wait_recv()
        for c in range(nc2):
            recv_desc(ry, c, dyr.at[c]).wait_recv()

        for c in range(nc2):
            recv_desc(diag, c, fyr.at[c]).wait_recv()
        for c in range(nc2, NC):
            recv_desc(diag, c, fzr.at[c - nc2]).wait_recv()

        for d in p1:
            d.wait_send()
        for d in st:
            d.wait()
        for d in dy + dz + fwd:
            d.wait_send()

    return pl.pallas_call(
        body,
        out_shape=jax.ShapeDtypeStruct((m, n), jnp.float32),
        in_specs=[pl.BlockSpec(memory_space=pltpu.MemorySpace.HBM)],
        out_specs=pl.BlockSpec(memory_space=pltpu.MemorySpace.HBM),
        scratch_shapes=[
            pltpu.VMEM((qsize, n), jnp.float32),
            pltpu.VMEM((qsize, n), jnp.float32),
            pltpu.SemaphoreType.DMA((NC,)),
            pltpu.SemaphoreType.DMA((NC,)),
            pltpu.SemaphoreType.DMA((NC,)),
            pltpu.SemaphoreType.DMA((NC,)),
            pltpu.SemaphoreType.DMA((NC,)),
            pltpu.SemaphoreType.DMA((NC,)),
            pltpu.SemaphoreType.DMA((NC // 2,)),
            pltpu.SemaphoreType.DMA((NC // 2,)),
            pltpu.SemaphoreType.DMA((NC // 2,)),
            pltpu.SemaphoreType.DMA((NC // 2,)),
            pltpu.SemaphoreType.DMA((NC,)),
            pltpu.SemaphoreType.DMA,
        ],
    )(x)
